# baseline (speedup 1.0000x reference)
import functools

import jax
import jax.numpy as jnp
from jax import lax
from jax.experimental import pallas as pl
from jax.experimental.pallas import tpu as pltpu
from jax.experimental.pallas import tpu_sc as plsc

_LN2 = 0.6931471805599453


def _ssp(v):
    return jnp.log(jnp.exp(v) + 1.0) - _LN2


def _stage1_body(AT, x_ref, w_ref, nbt_ref, y_ref, gidx_ref):
    b = pl.program_id(0)
    y_ref[0] = jnp.dot(x_ref[0].astype(jnp.bfloat16), w_ref[...],
                       preferred_element_type=jnp.float32)
    gidx_ref[0] = nbt_ref[0].T + b * AT


def _main_body(A_T, NBR, f_ref, mk_ref, x_ref, ynbr_ref,
               wf1_ref, bf1_ref, wf2_ref, bf2_ref, wfo_ref, bfo_ref,
               wd_ref, bd_ref, o_ref):
    E = A_T * NBR
    F = wf1_ref.shape[1]
    f = f_ref[0].reshape(E, -1).astype(jnp.bfloat16)
    h = jnp.dot(f, wf1_ref[...],
                preferred_element_type=jnp.float32).astype(jnp.bfloat16)
    h = _ssp(h + bf1_ref[...])
    w = jnp.dot(h, wf2_ref[...], preferred_element_type=jnp.float32)
    w = w + bf2_ref[...]
    z = w * ynbr_ref[0, 0]
    z = z.reshape(A_T, NBR, F) * mk_ref[0][:, :, None]
    agg = z.sum(axis=1)
    v = _ssp(jnp.dot(agg.astype(jnp.bfloat16), wfo_ref[...],
                     preferred_element_type=jnp.float32) + bfo_ref[...])
    out = jnp.dot(v.astype(jnp.bfloat16), wd_ref[...],
                  preferred_element_type=jnp.float32)
    o_ref[0] = out + bd_ref[...] + x_ref[0]


def _make_sc_gather(rows, F):
    info = plsc.get_sparse_core_info()
    nw = info.num_cores * info.num_subcores
    per_w = rows // nw
    chunk = 128
    nch = per_w // chunk
    mesh = plsc.VectorSubcoreMesh(core_axis_name="c", subcore_axis_name="s")

    @functools.partial(
        pl.kernel, mesh=mesh,
        out_type=jax.ShapeDtypeStruct((rows, F), jnp.float32),
        scratch_types=[
            pltpu.VMEM((chunk,), jnp.int32),
            pltpu.VMEM((chunk, F), jnp.float32),
            pltpu.SemaphoreType.DMA,
        ],
    )
    def gather_k(table_hbm, idx_hbm, out_hbm, idx_v, rows_v, sem):
        wid = lax.axis_index("s") * info.num_cores + lax.axis_index("c")
        base = wid * per_w

        def body(i, carry):
            off = base + i * chunk
            pltpu.sync_copy(idx_hbm.at[pl.ds(off, chunk)], idx_v)
            pltpu.async_copy(table_hbm.at[idx_v], rows_v, sem).wait()
            pltpu.sync_copy(rows_v, out_hbm.at[pl.ds(off, chunk)])
            return carry

        lax.fori_loop(0, nch, body, 0)

    return gather_k


def kernel(x, f_double, neighbors, neighbor_mask, Wf1, bf1, Wf2, bf2,
           W_in2f, W_f2out, b_f2out, W_dense, b_dense):
    B, AT, NBR = neighbors.shape
    G = f_double.shape[-1]
    F = Wf1.shape[1]
    NAB = x.shape[-1]
    A_T = 32
    nT = AT // A_T
    E = A_T * NBR
    ROWS = B * AT * NBR

    nbt = jnp.transpose(neighbors, (0, 2, 1))

    y, gidx = pl.pallas_call(
        functools.partial(_stage1_body, AT),
        grid=(B,),
        in_specs=[
            pl.BlockSpec((1, AT, NAB), lambda b: (b, 0, 0)),
            pl.BlockSpec((NAB, F), lambda b: (0, 0)),
            pl.BlockSpec((1, NBR, AT), lambda b: (b, 0, 0)),
        ],
        out_specs=[
            pl.BlockSpec((1, AT, F), lambda b: (b, 0, 0)),
            pl.BlockSpec((1, AT, NBR), lambda b: (b, 0, 0)),
        ],
        out_shape=[
            jax.ShapeDtypeStruct((B, AT, F), jnp.float32),
            jax.ShapeDtypeStruct((B, AT, NBR), jnp.int32),
        ],
    )(x, W_in2f.astype(jnp.bfloat16), nbt)

    y_nbr = _make_sc_gather(ROWS, F)(
        y.reshape(B * AT, F), gidx.reshape(ROWS))
    ynbr4 = y_nbr.reshape(B, nT, E, F)

    out = pl.pallas_call(
        functools.partial(_main_body, A_T, NBR),
        grid=(B, nT),
        in_specs=[
            pl.BlockSpec((1, A_T, NBR, G), lambda b, t: (b, t, 0, 0)),
            pl.BlockSpec((1, A_T, NBR), lambda b, t: (b, t, 0)),
            pl.BlockSpec((1, A_T, NAB), lambda b, t: (b, t, 0)),
            pl.BlockSpec((1, 1, E, F), lambda b, t: (b, t, 0, 0)),
            pl.BlockSpec((G, F), lambda b, t: (0, 0)),
            pl.BlockSpec((1, F), lambda b, t: (0, 0)),
            pl.BlockSpec((F, F), lambda b, t: (0, 0)),
            pl.BlockSpec((1, F), lambda b, t: (0, 0)),
            pl.BlockSpec((F, NAB), lambda b, t: (0, 0)),
            pl.BlockSpec((1, NAB), lambda b, t: (0, 0)),
            pl.BlockSpec((NAB, NAB), lambda b, t: (0, 0)),
            pl.BlockSpec((1, NAB), lambda b, t: (0, 0)),
        ],
        out_specs=pl.BlockSpec((1, A_T, NAB), lambda b, t: (b, t, 0)),
        out_shape=jax.ShapeDtypeStruct((B, AT, NAB), jnp.float32),
    )(f_double, neighbor_mask, x, ynbr4,
      Wf1.astype(jnp.bfloat16), bf1.astype(jnp.bfloat16).reshape(1, F),
      Wf2.astype(jnp.bfloat16), bf2.reshape(1, F),
      W_f2out.astype(jnp.bfloat16), b_f2out.reshape(1, NAB),
      W_dense.astype(jnp.bfloat16), b_dense.reshape(1, NAB))
    return out
